# baseline (device time: 157380 ns/iter reference)
import jax
import jax.numpy as jnp
from jax import lax
from jax.experimental import pallas as pl
from jax.experimental.pallas import tpu as pltpu

N = 16
NZ = 4
NP = 4
SQ = 256
D = 1024
HL = 8
DH = 128
SCALE = 0.08838834764831843
BF16 = jnp.bfloat16
F32 = jnp.float32


def _body(
    x_ref, wq_ref, wk_ref, wv_ref, wo_ref, out_ref,
    xg, ps, part_send, rsp_recv_buf, ps_send, rsz_recv_buf,
    col_send, col_recv, plane_send, plane_recv,
    rsp_send, rsp_recv, rsz_send, rsz_recv,
):
    my = lax.axis_index("i")
    myz = my // NP
    myp = my % NP

    def _rdma(src, dst, ssem, rsem, tgt):
        return pltpu.make_async_remote_copy(
            src_ref=src, dst_ref=dst, send_sem=ssem, recv_sem=rsem,
            device_id=(tgt,), device_id_type=pl.DeviceIdType.MESH,
        )

    def _partial(xb):
        qb = jnp.dot(xb, wq_ref[...], preferred_element_type=F32).astype(BF16)
        kb = jnp.dot(xb, wk_ref[...], preferred_element_type=F32).astype(BF16)
        vb = jnp.dot(xb, wv_ref[...], preferred_element_type=F32).astype(BF16)
        outs = []
        for h in range(HL):
            sl = slice(h * DH, (h + 1) * DH)
            s = lax.dot_general(
                qb[:, sl], kb[:, sl], (((1,), (1,)), ((), ())),
                preferred_element_type=F32,
            )
            pb = jnp.exp(s.astype(BF16))
            l = jnp.sum(pb, axis=1, dtype=F32, keepdims=True)
            outs.append(lax.dot_general(
                pb, vb[:, sl], (((1,), (0,)), ((), ())),
                preferred_element_type=F32,
            ) * (1.0 / l))
        attn = jnp.concatenate(outs, axis=1).astype(BF16)
        return jnp.dot(attn, wo_ref[...], preferred_element_type=F32)

    xg[my] = x_ref[...].astype(BF16)
    for k in (1, 2, 3):
        tgt = NP * ((myz + k) % NZ) + myp
        _rdma(xg.at[my], xg.at[my],
              col_send.at[k - 1], col_recv.at[NZ - k], tgt).start()
    for j in range(3):
        tgt = NP * myz + (myp + 1 + j) % NP
        _rdma(xg.at[my], xg.at[my],
              plane_send.at[0, j], plane_recv.at[0, 2 - j], tgt).start()

    ps[0] = _partial(xg[my])

    for o in (1, 2, 3):
        b = NP * ((myz + o) % NZ) + myp
        _rdma(xg.at[my], xg.at[b], col_send.at[0], col_recv.at[o],
              my).wait_recv()
        for j in range(3):
            tgt = NP * myz + (myp + 1 + j) % NP
            _rdma(xg.at[b], xg.at[b],
                  plane_send.at[o, j], plane_recv.at[o, 2 - j], tgt).start()
        ps[o] = _partial(xg[b])

    for j in range(3):
        agg = NP * myz + (myp + 1 + j) % NP
        for o in range(4):
            b = NP * ((myz + o) % NZ) + (myp + 1 + j) % NP
            _rdma(xg.at[my], xg.at[b], col_send.at[0], plane_recv.at[o, j],
                  my).wait_recv()
            part_send[j, o] = _partial(xg[b]).astype(BF16)
        _rdma(part_send.at[j], rsp_recv_buf.at[2 - j],
              rsp_send.at[j], rsp_recv.at[2 - j], agg).start()

    for j in (2, 1, 0):
        _rdma(xg.at[my], rsp_recv_buf.at[j], col_send.at[0],
              rsp_recv.at[j], my).wait_recv()
        for o in range(4):
            ps[o] = ps[o] + rsp_recv_buf[j, o].astype(F32)
    for o in (1, 2, 3):
        ps_send[o - 1] = ps[o].astype(BF16)
        tgt = NP * ((myz + o) % NZ) + myp
        _rdma(ps_send.at[o - 1], rsz_recv_buf.at[3 - o],
              rsz_send.at[o - 1], rsz_recv.at[NZ - o], tgt).start()

    acc = ps[0]
    for orr in (3, 2, 1):
        _rdma(xg.at[my], rsz_recv_buf.at[orr - 1], col_send.at[0],
              rsz_recv.at[orr], my).wait_recv()
        acc = acc + rsz_recv_buf[orr - 1].astype(F32)
    out_ref[...] = acc

    for k in (1, 2, 3):
        _rdma(xg.at[my], xg.at[my], col_send.at[k - 1], col_recv.at[0],
              my).wait_send()
    for o in range(4):
        for j in range(3):
            _rdma(xg.at[my], xg.at[my], plane_send.at[o, j],
                  plane_recv.at[0, 0], my).wait_send()
    for j in range(3):
        _rdma(part_send.at[j], rsp_recv_buf.at[j],
              rsp_send.at[j], rsp_recv.at[0], my).wait_send()
    for o in (1, 2, 3):
        _rdma(ps_send.at[o - 1], rsz_recv_buf.at[o - 1],
              rsz_send.at[o - 1], rsz_recv.at[0], my).wait_send()


def kernel(x, Wq, Wo, Wk, Wv):
    x2 = x.reshape(SQ, D)
    wq = (Wq * SCALE).astype(BF16)
    wk, wv, wo = (w.astype(BF16) for w in (Wk, Wv, Wo))

    out = pl.pallas_call(
        _body,
        out_shape=jax.ShapeDtypeStruct((SQ, D), F32),
        in_specs=[pl.BlockSpec(memory_space=pltpu.VMEM)] * 5,
        out_specs=pl.BlockSpec(memory_space=pltpu.VMEM),
        scratch_shapes=[
            pltpu.VMEM((N, SQ, D), BF16),
            pltpu.VMEM((NZ, SQ, D), F32),
            pltpu.VMEM((3, NZ, SQ, D), BF16),
            pltpu.VMEM((3, NZ, SQ, D), BF16),
            pltpu.VMEM((3, SQ, D), BF16),
            pltpu.VMEM((3, SQ, D), BF16),
            pltpu.SemaphoreType.DMA((3,)),
            pltpu.SemaphoreType.DMA((NZ,)),
            pltpu.SemaphoreType.DMA((NZ, 3)),
            pltpu.SemaphoreType.DMA((NZ, 3)),
            pltpu.SemaphoreType.DMA((3,)),
            pltpu.SemaphoreType.DMA((3,)),
            pltpu.SemaphoreType.DMA((3,)),
            pltpu.SemaphoreType.DMA((NZ,)),
        ],
    )(x2, wq, wk, wv, wo)

    return out.reshape(1, SQ, D)


# device time: 141942 ns/iter; 1.1088x vs baseline; 1.1088x over previous
import jax
import jax.numpy as jnp
from jax import lax
from jax.experimental import pallas as pl
from jax.experimental.pallas import tpu as pltpu

N = 16
NZ = 4
NP = 4
SQ = 256
D = 1024
HL = 8
DH = 128
SCALE = 0.08838834764831843
BF16 = jnp.bfloat16
F32 = jnp.float32


def _body(
    x_ref, wq_ref, wk_ref, wv_ref, wo_ref, out_ref,
    xg, ps, part_send, rsp_recv_buf, ps_send, rsz_recv_buf,
    col_send, col_recv, plane_send, plane_recv,
    rsp_send, rsp_recv, rsz_send, rsz_recv,
):
    my = lax.axis_index("i")
    myz = my // NP
    myp = my % NP

    def _rdma(src, dst, ssem, rsem, tgt):
        return pltpu.make_async_remote_copy(
            src_ref=src, dst_ref=dst, send_sem=ssem, recv_sem=rsem,
            device_id=(tgt,), device_id_type=pl.DeviceIdType.MESH,
        )

    def _partial(xb):
        qb = jnp.dot(xb, wq_ref[...], preferred_element_type=F32).astype(BF16)
        kb = jnp.dot(xb, wk_ref[...], preferred_element_type=F32).astype(BF16)
        vb = jnp.dot(xb, wv_ref[...], preferred_element_type=F32).astype(BF16)
        outs = []
        for h in range(HL):
            sl = slice(h * DH, (h + 1) * DH)
            s = lax.dot_general(
                qb[:, sl], kb[:, sl], (((1,), (1,)), ((), ())),
                preferred_element_type=F32,
            )
            pb = jnp.exp(s.astype(BF16))
            l = jnp.sum(pb, axis=1, dtype=F32, keepdims=True)
            outs.append(lax.dot_general(
                pb, vb[:, sl], (((1,), (0,)), ((), ())),
                preferred_element_type=F32,
            ) * (1.0 / l))
        attn = jnp.concatenate(outs, axis=1).astype(BF16)
        return jnp.dot(attn, wo_ref[...], preferred_element_type=F32)

    xg[my] = x_ref[...].astype(BF16)
    for k in (1, 2, 3):
        tgt = NP * ((myz + k) % NZ) + myp
        _rdma(xg.at[my], xg.at[my],
              col_send.at[k - 1], col_recv.at[NZ - k], tgt).start()
    for j in range(3):
        tgt = NP * myz + (myp + 1 + j) % NP
        _rdma(xg.at[my], xg.at[my],
              plane_send.at[0, j], plane_recv.at[0, 2 - j], tgt).start()

    ps[0] = _partial(xg[my])

    for o in (1, 2, 3):
        b = NP * ((myz + o) % NZ) + myp
        _rdma(xg.at[my], xg.at[b], col_send.at[0], col_recv.at[o],
              my).wait_recv()
        for j in range(3):
            tgt = NP * myz + (myp + 1 + j) % NP
            _rdma(xg.at[b], xg.at[b],
                  plane_send.at[o, j], plane_recv.at[o, 2 - j], tgt).start()
        ps[o] = _partial(xg[b])

    for j in range(3):
        agg = NP * myz + (myp + 1 + j) % NP
        for o in range(4):
            b = NP * ((myz + o) % NZ) + (myp + 1 + j) % NP
            _rdma(xg.at[my], xg.at[b], col_send.at[0], plane_recv.at[o, j],
                  my).wait_recv()
            part_send[o, j] = _partial(xg[b]).astype(BF16)
            _rdma(part_send.at[o, j], rsp_recv_buf.at[o, 2 - j],
                  rsp_send.at[o, j], rsp_recv.at[o, 2 - j], agg).start()

    for o in (1, 2, 3, 0):
        acc = ps[o]
        for j in (2, 1, 0):
            _rdma(xg.at[my], rsp_recv_buf.at[o, j], col_send.at[0],
                  rsp_recv.at[o, j], my).wait_recv()
            acc = acc + rsp_recv_buf[o, j].astype(F32)
        if o == 0:
            ps[0] = acc
        else:
            ps_send[o - 1] = acc.astype(BF16)
            tgt = NP * ((myz + o) % NZ) + myp
            _rdma(ps_send.at[o - 1], rsz_recv_buf.at[3 - o],
                  rsz_send.at[o - 1], rsz_recv.at[NZ - o], tgt).start()

    acc = ps[0]
    for orr in (3, 2, 1):
        _rdma(xg.at[my], rsz_recv_buf.at[orr - 1], col_send.at[0],
              rsz_recv.at[orr], my).wait_recv()
        acc = acc + rsz_recv_buf[orr - 1].astype(F32)
    out_ref[...] = acc

    for k in (1, 2, 3):
        _rdma(xg.at[my], xg.at[my], col_send.at[k - 1], col_recv.at[0],
              my).wait_send()
    for o in range(4):
        for j in range(3):
            _rdma(xg.at[my], xg.at[my], plane_send.at[o, j],
                  plane_recv.at[0, 0], my).wait_send()
            _rdma(part_send.at[o, j], rsp_recv_buf.at[o, j],
                  rsp_send.at[o, j], rsp_recv.at[0, 0], my).wait_send()
    for o in (1, 2, 3):
        _rdma(ps_send.at[o - 1], rsz_recv_buf.at[o - 1],
              rsz_send.at[o - 1], rsz_recv.at[0], my).wait_send()


def kernel(x, Wq, Wo, Wk, Wv):
    x2 = x.reshape(SQ, D)
    wq = (Wq * SCALE).astype(BF16)
    wk, wv, wo = (w.astype(BF16) for w in (Wk, Wv, Wo))

    out = pl.pallas_call(
        _body,
        out_shape=jax.ShapeDtypeStruct((SQ, D), F32),
        in_specs=[pl.BlockSpec(memory_space=pltpu.VMEM)] * 5,
        out_specs=pl.BlockSpec(memory_space=pltpu.VMEM),
        scratch_shapes=[
            pltpu.VMEM((N, SQ, D), BF16),
            pltpu.VMEM((NZ, SQ, D), F32),
            pltpu.VMEM((NZ, 3, SQ, D), BF16),
            pltpu.VMEM((NZ, 3, SQ, D), BF16),
            pltpu.VMEM((3, SQ, D), BF16),
            pltpu.VMEM((3, SQ, D), BF16),
            pltpu.SemaphoreType.DMA((3,)),
            pltpu.SemaphoreType.DMA((NZ,)),
            pltpu.SemaphoreType.DMA((NZ, 3)),
            pltpu.SemaphoreType.DMA((NZ, 3)),
            pltpu.SemaphoreType.DMA((NZ, 3)),
            pltpu.SemaphoreType.DMA((NZ, 3)),
            pltpu.SemaphoreType.DMA((3,)),
            pltpu.SemaphoreType.DMA((NZ,)),
        ],
    )(x2, wq, wk, wv, wo)

    return out.reshape(1, SQ, D)
